# Initial kernel scaffold; baseline (speedup 1.0000x reference)
import jax
import jax.numpy as jnp
from jax.experimental import pallas as pl


def kernel(u_f, v_f, edge_index):
    raise NotImplementedError("write your pallas kernel here")


# speedup vs baseline: 5.5795x; 5.5795x over previous
import functools

import jax
import jax.numpy as jnp
from jax import lax
from jax.experimental import pallas as pl
from jax.experimental.pallas import tpu as pltpu
from jax.experimental.pallas import tpu_sc as plsc

N_NODES = 10000
NP = 10240
E = 320000
D = 128
NC, NS, L = 2, 16, 16
NW = NC * NS
EPT = E // NW
CHUNK = 80
NCH = EPT // CHUNK
ROWS_PT = NP // NS

_mesh = functools.partial(
    plsc.VectorSubcoreMesh, core_axis_name="c", subcore_axis_name="s")


def _hist_body(edges_hbm, hist_hbm, src_v, dst_v, hout_v, hin_v):
    cid = lax.axis_index("c")
    sid = lax.axis_index("s")
    wid = sid * NC + cid
    pltpu.sync_copy(edges_hbm.at[0, wid], src_v)
    pltpu.sync_copy(edges_hbm.at[1, wid], dst_v)

    zeros16 = jnp.zeros((L,), jnp.float32)
    ones16 = jnp.ones((L,), jnp.float32)

    def zero(i, c):
        hout_v[pl.ds(i * L, L)] = zeros16
        hin_v[pl.ds(i * L, L)] = zeros16
        return c

    lax.fori_loop(0, NP // L, zero, 0)

    def accum(r, c):
        for k in range(CHUNK // L):
            sidx = src_v[r, pl.ds(k * L, L)]
            plsc.addupdate_scatter(hout_v, [sidx], ones16)
            didx = dst_v[r, pl.ds(k * L, L)]
            plsc.addupdate_scatter(hin_v, [didx], ones16)
        return c

    lax.fori_loop(0, NCH, accum, 0)

    pltpu.sync_copy(hout_v, hist_hbm.at[0, wid, 0])
    pltpu.sync_copy(hin_v, hist_hbm.at[1, wid, 0])


def _hist_call(edges):
    return pl.kernel(
        _hist_body,
        out_type=jax.ShapeDtypeStruct((2, NW, 1, NP), jnp.float32),
        mesh=_mesh(),
        scratch_types=[
            pltpu.VMEM((NCH, CHUNK), jnp.int32),
            pltpu.VMEM((NCH, CHUNK), jnp.int32),
            pltpu.VMEM((NP,), jnp.float32),
            pltpu.VMEM((NP,), jnp.float32),
        ],
        compiler_params=pltpu.CompilerParams(needs_layout_passes=False),
    )(edges)


def _scale_body(hist_ref, nf_ref, out_ref):
    deg = jnp.maximum(jnp.sum(hist_ref[0], axis=0), 1.0)
    norm = lax.rsqrt(deg)
    rows = lax.broadcasted_iota(jnp.int32, (128, 128), 0)
    cols = lax.broadcasted_iota(jnp.int32, (128, 128), 1)
    diag = jnp.where(rows == cols, norm[None, :], 0.0)
    out_ref[...] = jnp.dot(diag, nf_ref[...],
                           preferred_element_type=jnp.float32,
                           precision=lax.Precision.HIGHEST)


def _scale_call(hist, node_f):
    return pl.pallas_call(
        _scale_body,
        grid=(NP // 128,),
        in_specs=[
            pl.BlockSpec((2, NW, 128), lambda j: (0, 0, j)),
            pl.BlockSpec((128, D), lambda j: (j, 0)),
        ],
        out_specs=pl.BlockSpec((128, D), lambda j: (j, 0)),
        out_shape=jax.ShapeDtypeStruct((NP, D), jnp.float32),
    )(hist, node_f)


def _mp_body(scaled_hbm, edges_hbm, acc_hbm, src_v, dst_v, rows_v, gsem,
             acc_sh):
    cid = lax.axis_index("c")
    sid = lax.axis_index("s")
    wid = sid * NC + cid
    pltpu.sync_copy(edges_hbm.at[0, wid], src_v)
    pltpu.sync_copy(edges_hbm.at[1, wid], dst_v)

    zeros16 = jnp.zeros((L,), jnp.float32)

    def zero(r, c):
        for k in range(D // L):
            rows_v[r, pl.ds(k * L, L)] = zeros16
        return c

    lax.fori_loop(0, CHUNK, zero, 0)
    for b in range(ROWS_PT // CHUNK):
        pltpu.sync_copy(
            rows_v, acc_sh.at[pl.ds(sid * ROWS_PT + b * CHUNK, CHUNK)])
    plsc.subcore_barrier()

    def step(j, c):
        pltpu.async_copy(scaled_hbm.at[src_v.at[j]], rows_v, gsem).wait()
        pltpu.sync_copy(rows_v, acc_sh.at[dst_v.at[j]], add=True)
        return c

    lax.fori_loop(0, NCH, step, 0)

    plsc.subcore_barrier()
    pltpu.sync_copy(acc_sh.at[pl.ds(sid * ROWS_PT, ROWS_PT)],
                    acc_hbm.at[cid, pl.ds(sid * ROWS_PT, ROWS_PT)])


def _mp_call(scaled, edges):
    return pl.kernel(
        _mp_body,
        out_type=jax.ShapeDtypeStruct((NC, NP, D), jnp.float32),
        mesh=_mesh(),
        scratch_types=[
            pltpu.VMEM((NCH, CHUNK), jnp.int32),
            pltpu.VMEM((NCH, CHUNK), jnp.int32),
            pltpu.VMEM((CHUNK, D), jnp.float32),
            pltpu.SemaphoreType.DMA,
            pltpu.VMEM_SHARED((NP, D), jnp.float32),
        ],
    )(scaled, edges)


def _final_body(acc_ref, hist_ref, out_ref):
    deg = jnp.maximum(jnp.sum(hist_ref[0], axis=0), 1.0)
    norm = lax.rsqrt(deg)
    rows = lax.broadcasted_iota(jnp.int32, (128, 128), 0)
    cols = lax.broadcasted_iota(jnp.int32, (128, 128), 1)
    diag = jnp.where(rows == cols, norm[None, :], 0.0)
    total = acc_ref[0] + acc_ref[1]
    out_ref[...] = jnp.dot(diag, total,
                           preferred_element_type=jnp.float32,
                           precision=lax.Precision.HIGHEST)


def _final_call(acc, hist):
    return pl.pallas_call(
        _final_body,
        grid=(NP // 128,),
        in_specs=[
            pl.BlockSpec((NC, 128, D), lambda j: (0, j, 0)),
            pl.BlockSpec((1, NW, 128), lambda j: (1, 0, j)),
        ],
        out_specs=pl.BlockSpec((128, D), lambda j: (j, 0)),
        out_shape=jax.ShapeDtypeStruct((NP, D), jnp.float32),
    )(acc, hist)


def kernel(u_f, v_f, edge_index):
    node_f = jnp.concatenate([u_f, v_f], axis=0)
    node_f = jnp.pad(node_f, ((0, NP - N_NODES), (0, 0)))
    edges = edge_index.reshape(2, NW, NCH, CHUNK)
    hist = _hist_call(edges).reshape(2, NW, NP)
    scaled = _scale_call(hist, node_f)
    acc = _mp_call(scaled, edges)
    out = _final_call(acc, hist)
    return out[:N_NODES]


# speedup vs baseline: 7.4750x; 1.3397x over previous
import functools

import jax
import jax.numpy as jnp
from jax import lax
from jax.experimental import pallas as pl
from jax.experimental.pallas import tpu as pltpu
from jax.experimental.pallas import tpu_sc as plsc

N_NODES = 10000
NP = 10240
E = 320000
D = 128
NC, NS, L = 2, 16, 16
NW = NC * NS
EPT = E // NW
CHUNK = 80
NCH = EPT // CHUNK
NCH_H = EPT // L
ROWS_PT = NP // NS

_mesh = functools.partial(
    plsc.VectorSubcoreMesh, core_axis_name="c", subcore_axis_name="s")


def _hist_body(edges_hbm, hist_hbm, src_v, dst_v, hout_v, hin_v):
    cid = lax.axis_index("c")
    sid = lax.axis_index("s")
    wid = sid * NC + cid
    pltpu.sync_copy(edges_hbm.at[0, wid, 0], src_v)
    pltpu.sync_copy(edges_hbm.at[1, wid, 0], dst_v)

    zeros16 = jnp.zeros((L,), jnp.float32)
    ones16 = jnp.ones((L,), jnp.float32)

    def zero(i, c):
        hout_v[pl.ds(i * L, L)] = zeros16
        hin_v[pl.ds(i * L, L)] = zeros16
        return c

    lax.fori_loop(0, NP // L, zero, 0)

    def accum(r, c):
        sidx = src_v[pl.ds(r * L, L)]
        plsc.addupdate_scatter(hout_v, [sidx], ones16)
        didx = dst_v[pl.ds(r * L, L)]
        plsc.addupdate_scatter(hin_v, [didx], ones16)
        return c

    lax.fori_loop(0, NCH_H, accum, 0)

    pltpu.sync_copy(hout_v, hist_hbm.at[0, wid, 0])
    pltpu.sync_copy(hin_v, hist_hbm.at[1, wid, 0])


def _hist_call(edges):
    return pl.kernel(
        _hist_body,
        out_type=jax.ShapeDtypeStruct((2, NW, 1, NP), jnp.float32),
        mesh=_mesh(),
        scratch_types=[
            pltpu.VMEM((EPT,), jnp.int32),
            pltpu.VMEM((EPT,), jnp.int32),
            pltpu.VMEM((NP,), jnp.float32),
            pltpu.VMEM((NP,), jnp.float32),
        ],
        compiler_params=pltpu.CompilerParams(needs_layout_passes=False),
    )(edges)


def _scale_body(hist_ref, nf_ref, out_ref):
    deg = jnp.maximum(jnp.sum(hist_ref[0], axis=0), 1.0)
    norm = lax.rsqrt(deg)
    rows = lax.broadcasted_iota(jnp.int32, (128, 128), 0)
    cols = lax.broadcasted_iota(jnp.int32, (128, 128), 1)
    diag = jnp.where(rows == cols, norm[None, :], 0.0)
    out_ref[...] = jnp.dot(diag, nf_ref[...],
                           preferred_element_type=jnp.float32,
                           precision=lax.Precision.HIGHEST)


def _scale_call(hist, node_f):
    return pl.pallas_call(
        _scale_body,
        grid=(NP // 128,),
        in_specs=[
            pl.BlockSpec((2, NW, 128), lambda j: (0, 0, j)),
            pl.BlockSpec((128, D), lambda j: (j, 0)),
        ],
        out_specs=pl.BlockSpec((128, D), lambda j: (j, 0)),
        out_shape=jax.ShapeDtypeStruct((NP, D), jnp.float32),
    )(hist, node_f)


NBUF = 2


def _mp_body(scaled_hbm, eflat_hbm, edges_hbm, acc_hbm, src_v, dst_v, rows_v,
             gsem, ssem, acc_sh):
    cid = lax.axis_index("c")
    sid = lax.axis_index("s")
    wid = sid * NC + cid
    pltpu.sync_copy(eflat_hbm.at[0, wid, 0], src_v)
    pltpu.sync_copy(edges_hbm.at[1, wid], dst_v)

    zeros16 = jnp.zeros((L,), jnp.float32)

    def zero(r, c):
        for k in range(D // L):
            rows_v[0, r, pl.ds(k * L, L)] = zeros16
        return c

    lax.fori_loop(0, CHUNK, zero, 0)
    for b in range(ROWS_PT // CHUNK):
        pltpu.sync_copy(
            rows_v.at[0], acc_sh.at[pl.ds(sid * ROWS_PT + b * CHUNK, CHUNK)])
    plsc.subcore_barrier()

    def g_start(j, b):
        pltpu.async_copy(scaled_hbm.at[src_v.at[pl.ds(j * CHUNK, CHUNK)]],
                         rows_v.at[b], gsem.at[b])

    def g_wait(j, b):
        pltpu.make_async_copy(scaled_hbm.at[src_v.at[pl.ds(j * CHUNK, CHUNK)]],
                              rows_v.at[b], gsem.at[b]).wait()

    def s_start(j, b):
        pltpu.async_copy(rows_v.at[b], acc_sh.at[dst_v.at[j]],
                         ssem.at[b], add=True)

    def s_wait(j, b):
        pltpu.make_async_copy(rows_v.at[b], acc_sh.at[dst_v.at[j]],
                              ssem.at[b]).wait()

    g_start(0, 0)
    g_start(1, 1)

    def step(i, c):
        for b in range(NBUF):
            j = i * NBUF + b
            g_wait(j, b)
            s_start(j, b)
            s_wait(j, b)
            if b == 0:
                g_start(j + NBUF, b)
            else:
                @pl.when(i < NCH // NBUF - 1)
                def _():
                    g_start(j + NBUF, b)
        return c

    lax.fori_loop(0, NCH // NBUF, step, 0)
    g_wait(NCH - 1, 0)
    s_start(NCH - 1, 0)
    s_wait(NCH - 1, 0)

    plsc.subcore_barrier()
    pltpu.sync_copy(acc_sh.at[pl.ds(sid * ROWS_PT, ROWS_PT)],
                    acc_hbm.at[cid, pl.ds(sid * ROWS_PT, ROWS_PT)])


def _mp_call(scaled, eflat, edges):
    return pl.kernel(
        _mp_body,
        out_type=jax.ShapeDtypeStruct((NC, NP, D), jnp.float32),
        mesh=_mesh(),
        scratch_types=[
            pltpu.VMEM((EPT,), jnp.int32),
            pltpu.VMEM((NCH, CHUNK), jnp.int32),
            pltpu.VMEM((NBUF, CHUNK, D), jnp.float32),
            pltpu.SemaphoreType.DMA((NBUF,)),
            pltpu.SemaphoreType.DMA((NBUF,)),
            pltpu.VMEM_SHARED((NP, D), jnp.float32),
        ],
    )(scaled, eflat, edges)


def _final_body(acc_ref, hist_ref, out_ref):
    deg = jnp.maximum(jnp.sum(hist_ref[0], axis=0), 1.0)
    norm = lax.rsqrt(deg)
    rows = lax.broadcasted_iota(jnp.int32, (128, 128), 0)
    cols = lax.broadcasted_iota(jnp.int32, (128, 128), 1)
    diag = jnp.where(rows == cols, norm[None, :], 0.0)
    total = acc_ref[0] + acc_ref[1]
    out_ref[...] = jnp.dot(diag, total,
                           preferred_element_type=jnp.float32,
                           precision=lax.Precision.HIGHEST)


def _final_call(acc, hist):
    return pl.pallas_call(
        _final_body,
        grid=(NP // 128,),
        in_specs=[
            pl.BlockSpec((NC, 128, D), lambda j: (0, j, 0)),
            pl.BlockSpec((1, NW, 128), lambda j: (1, 0, j)),
        ],
        out_specs=pl.BlockSpec((128, D), lambda j: (j, 0)),
        out_shape=jax.ShapeDtypeStruct((NP, D), jnp.float32),
    )(acc, hist)


def kernel(u_f, v_f, edge_index):
    node_f = jnp.concatenate([u_f, v_f], axis=0)
    node_f = jnp.pad(node_f, ((0, NP - N_NODES), (0, 0)))
    eflat = edge_index.reshape(2, NW, 1, EPT)
    edges = edge_index.reshape(2, NW, NCH, CHUNK)
    hist = _hist_call(eflat).reshape(2, NW, NP)
    scaled = _scale_call(hist, node_f)
    acc = _mp_call(scaled, eflat, edges)
    out = _final_call(acc, hist)
    return out[:N_NODES]


# speedup vs baseline: 7.6471x; 1.0230x over previous
import functools

import jax
import jax.numpy as jnp
from jax import lax
from jax.experimental import pallas as pl
from jax.experimental.pallas import tpu as pltpu
from jax.experimental.pallas import tpu_sc as plsc

N_NODES = 10000
NP = 10240
E = 320000
D = 128
NC, NS, L = 2, 16, 16
NW = NC * NS
EPT = E // NW
CHUNK = 80
NCH = EPT // CHUNK
NCH_H = EPT // L
ROWS_PT = NP // NS

_mesh = functools.partial(
    plsc.VectorSubcoreMesh, core_axis_name="c", subcore_axis_name="s")


def _hist_body(edges_hbm, hist_hbm, src_v, dst_v, hout_v, hin_v):
    cid = lax.axis_index("c")
    sid = lax.axis_index("s")
    wid = sid * NC + cid
    pltpu.sync_copy(edges_hbm.at[0, wid, 0], src_v)
    pltpu.sync_copy(edges_hbm.at[1, wid, 0], dst_v)

    zeros16 = jnp.zeros((L,), jnp.float32)
    ones16 = jnp.ones((L,), jnp.float32)

    def zero(i, c):
        hout_v[pl.ds(i * L, L)] = zeros16
        hin_v[pl.ds(i * L, L)] = zeros16
        return c

    lax.fori_loop(0, NP // L, zero, 0)

    def accum(r, c):
        sidx = src_v[pl.ds(r * L, L)]
        plsc.addupdate_scatter(hout_v, [sidx], ones16)
        didx = dst_v[pl.ds(r * L, L)]
        plsc.addupdate_scatter(hin_v, [didx], ones16)
        return c

    lax.fori_loop(0, NCH_H, accum, 0)

    pltpu.sync_copy(hout_v, hist_hbm.at[0, wid, 0])
    pltpu.sync_copy(hin_v, hist_hbm.at[1, wid, 0])


def _hist_call(edges):
    return pl.kernel(
        _hist_body,
        out_type=jax.ShapeDtypeStruct((2, NW, 1, NP), jnp.float32),
        mesh=_mesh(),
        scratch_types=[
            pltpu.VMEM((EPT,), jnp.int32),
            pltpu.VMEM((EPT,), jnp.int32),
            pltpu.VMEM((NP,), jnp.float32),
            pltpu.VMEM((NP,), jnp.float32),
        ],
        compiler_params=pltpu.CompilerParams(needs_layout_passes=False),
    )(edges)


def _scale_body(hist_ref, nf_ref, out_ref):
    deg = jnp.maximum(jnp.sum(hist_ref[0], axis=0), 1.0)
    norm = lax.rsqrt(deg)
    norm_col = jnp.broadcast_to(norm[None, :], (128, 128)).T
    out_ref[...] = nf_ref[...] * norm_col


def _scale_call(hist, node_f):
    return pl.pallas_call(
        _scale_body,
        grid=(NP // 128,),
        in_specs=[
            pl.BlockSpec((2, NW, 128), lambda j: (0, 0, j)),
            pl.BlockSpec((128, D), lambda j: (j, 0)),
        ],
        out_specs=pl.BlockSpec((128, D), lambda j: (j, 0)),
        out_shape=jax.ShapeDtypeStruct((NP, D), jnp.float32),
    )(hist, node_f)


NBUF = 2


def _mp_body(scaled_hbm, eflat_hbm, edges_hbm, acc_hbm, src_v, dst_v, rows_v,
             gsem, ssem, acc_sh):
    cid = lax.axis_index("c")
    sid = lax.axis_index("s")
    wid = sid * NC + cid
    pltpu.sync_copy(eflat_hbm.at[0, wid, 0], src_v)
    pltpu.sync_copy(edges_hbm.at[1, wid], dst_v)

    zeros16 = jnp.zeros((L,), jnp.float32)

    def zero(r, c):
        for k in range(D // L):
            rows_v[0, r, pl.ds(k * L, L)] = zeros16
        return c

    lax.fori_loop(0, CHUNK, zero, 0)
    for b in range(ROWS_PT // CHUNK):
        pltpu.sync_copy(
            rows_v.at[0], acc_sh.at[pl.ds(sid * ROWS_PT + b * CHUNK, CHUNK)])
    plsc.subcore_barrier()

    def g_start(j, b):
        pltpu.async_copy(scaled_hbm.at[src_v.at[pl.ds(j * CHUNK, CHUNK)]],
                         rows_v.at[b], gsem.at[b])

    def g_wait(j, b):
        pltpu.make_async_copy(scaled_hbm.at[src_v.at[pl.ds(j * CHUNK, CHUNK)]],
                              rows_v.at[b], gsem.at[b]).wait()

    def s_start(j, b):
        pltpu.async_copy(rows_v.at[b], acc_sh.at[dst_v.at[j]],
                         ssem.at[b], add=True)

    def s_wait(j, b):
        pltpu.make_async_copy(rows_v.at[b], acc_sh.at[dst_v.at[j]],
                              ssem.at[b]).wait()

    g_start(0, 0)
    g_start(1, 1)

    def step(i, c):
        for b in range(NBUF):
            j = i * NBUF + b
            g_wait(j, b)
            s_start(j, b)
            s_wait(j, b)
            if b == 0:
                g_start(j + NBUF, b)
            else:
                @pl.when(i < NCH // NBUF - 1)
                def _():
                    g_start(j + NBUF, b)
        return c

    lax.fori_loop(0, NCH // NBUF, step, 0)
    g_wait(NCH - 1, 0)
    s_start(NCH - 1, 0)
    s_wait(NCH - 1, 0)

    plsc.subcore_barrier()
    pltpu.sync_copy(acc_sh.at[pl.ds(sid * ROWS_PT, ROWS_PT)],
                    acc_hbm.at[cid, pl.ds(sid * ROWS_PT, ROWS_PT)])


def _mp_call(scaled, eflat, edges):
    return pl.kernel(
        _mp_body,
        out_type=jax.ShapeDtypeStruct((NC, NP, D), jnp.float32),
        mesh=_mesh(),
        scratch_types=[
            pltpu.VMEM((EPT,), jnp.int32),
            pltpu.VMEM((NCH, CHUNK), jnp.int32),
            pltpu.VMEM((NBUF, CHUNK, D), jnp.float32),
            pltpu.SemaphoreType.DMA((NBUF,)),
            pltpu.SemaphoreType.DMA((NBUF,)),
            pltpu.VMEM_SHARED((NP, D), jnp.float32),
        ],
    )(scaled, eflat, edges)


def _final_body(acc_ref, hist_ref, out_ref):
    deg = jnp.maximum(jnp.sum(hist_ref[0], axis=0), 1.0)
    norm = lax.rsqrt(deg)
    norm_col = jnp.broadcast_to(norm[None, :], (128, 128)).T
    out_ref[...] = (acc_ref[0] + acc_ref[1]) * norm_col


def _final_call(acc, hist):
    return pl.pallas_call(
        _final_body,
        grid=(NP // 128,),
        in_specs=[
            pl.BlockSpec((NC, 128, D), lambda j: (0, j, 0)),
            pl.BlockSpec((1, NW, 128), lambda j: (1, 0, j)),
        ],
        out_specs=pl.BlockSpec((128, D), lambda j: (j, 0)),
        out_shape=jax.ShapeDtypeStruct((NP, D), jnp.float32),
    )(acc, hist)


def kernel(u_f, v_f, edge_index):
    node_f = jnp.concatenate([u_f, v_f], axis=0)
    node_f = jnp.pad(node_f, ((0, NP - N_NODES), (0, 0)))
    eflat = edge_index.reshape(2, NW, 1, EPT)
    edges = edge_index.reshape(2, NW, NCH, CHUNK)
    hist = _hist_call(eflat).reshape(2, NW, NP)
    scaled = _scale_call(hist, node_f)
    acc = _mp_call(scaled, eflat, edges)
    out = _final_call(acc, hist)
    return out[:N_NODES]


# speedup vs baseline: 10.5853x; 1.3842x over previous
import functools

import jax
import jax.numpy as jnp
from jax import lax
from jax.experimental import pallas as pl
from jax.experimental.pallas import tpu as pltpu
from jax.experimental.pallas import tpu_sc as plsc

N_NODES = 10000
NP = 10240
E = 320000
D = 128
NC, NS, L = 2, 16, 16
NW = NC * NS
EPT = E // NW
CHUNK = 80
NCH = EPT // CHUNK
NCH_H = EPT // L
ROWS_PT = NP // NS

_mesh = functools.partial(
    plsc.VectorSubcoreMesh, core_axis_name="c", subcore_axis_name="s")


def _hist_body(edges_hbm, hist_hbm, src_v, dst_v, hout_v, hin_v):
    cid = lax.axis_index("c")
    sid = lax.axis_index("s")
    wid = sid * NC + cid
    pltpu.sync_copy(edges_hbm.at[0, wid, 0], src_v)
    pltpu.sync_copy(edges_hbm.at[1, wid, 0], dst_v)

    zeros16 = jnp.zeros((L,), jnp.float32)
    ones16 = jnp.ones((L,), jnp.float32)

    def zero(i, c):
        hout_v[pl.ds(i * L, L)] = zeros16
        hin_v[pl.ds(i * L, L)] = zeros16
        return c

    lax.fori_loop(0, NP // L, zero, 0)

    def accum(r, c):
        sidx = src_v[pl.ds(r * L, L)]
        plsc.addupdate_scatter(hout_v, [sidx], ones16)
        didx = dst_v[pl.ds(r * L, L)]
        plsc.addupdate_scatter(hin_v, [didx], ones16)
        return c

    lax.fori_loop(0, NCH_H, accum, 0)

    pltpu.sync_copy(hout_v, hist_hbm.at[0, wid, 0])
    pltpu.sync_copy(hin_v, hist_hbm.at[1, wid, 0])


def _hist_call(edges):
    return pl.kernel(
        _hist_body,
        out_type=jax.ShapeDtypeStruct((2, NW, 1, NP), jnp.float32),
        mesh=_mesh(),
        scratch_types=[
            pltpu.VMEM((EPT,), jnp.int32),
            pltpu.VMEM((EPT,), jnp.int32),
            pltpu.VMEM((NP,), jnp.float32),
            pltpu.VMEM((NP,), jnp.float32),
        ],
        compiler_params=pltpu.CompilerParams(needs_layout_passes=False),
    )(edges)


ROWS_TC = 1024


def _scale_body(hist_ref, nf_ref, out_ref):
    deg = jnp.maximum(jnp.sum(hist_ref[0], axis=0), 1.0)
    norm = lax.rsqrt(deg)
    for k in range(ROWS_TC // 128):
        sl = slice(k * 128, (k + 1) * 128)
        col = jnp.broadcast_to(norm[None, sl], (128, 128)).T
        out_ref[sl, :] = nf_ref[sl, :] * col


def _scale_call(hist, node_f):
    return pl.pallas_call(
        _scale_body,
        grid=(NP // ROWS_TC,),
        in_specs=[
            pl.BlockSpec((2, NW, ROWS_TC), lambda j: (0, 0, j)),
            pl.BlockSpec((ROWS_TC, D), lambda j: (j, 0)),
        ],
        out_specs=pl.BlockSpec((ROWS_TC, D), lambda j: (j, 0)),
        out_shape=jax.ShapeDtypeStruct((NP, D), jnp.float32),
    )(hist, node_f)


NBUF = 2


def _mp_body(scaled_hbm, eflat_hbm, edges_hbm, acc_hbm, src_v, dst_v, rows_v,
             gsem, ssem, acc_sh):
    cid = lax.axis_index("c")
    sid = lax.axis_index("s")
    wid = sid * NC + cid
    pltpu.sync_copy(eflat_hbm.at[0, wid, 0], src_v)
    pltpu.sync_copy(edges_hbm.at[1, wid], dst_v)

    zeros16 = jnp.zeros((L,), jnp.float32)

    def zero(r, c):
        for k in range(D // L):
            rows_v[0, r, pl.ds(k * L, L)] = zeros16
        return c

    lax.fori_loop(0, CHUNK, zero, 0)
    for b in range(ROWS_PT // CHUNK):
        pltpu.sync_copy(
            rows_v.at[0], acc_sh.at[pl.ds(sid * ROWS_PT + b * CHUNK, CHUNK)])
    plsc.subcore_barrier()

    def g_start(j, b):
        pltpu.async_copy(scaled_hbm.at[src_v.at[pl.ds(j * CHUNK, CHUNK)]],
                         rows_v.at[b], gsem.at[b])

    def g_wait(j, b):
        pltpu.make_async_copy(scaled_hbm.at[src_v.at[pl.ds(j * CHUNK, CHUNK)]],
                              rows_v.at[b], gsem.at[b]).wait()

    def s_start(j, b):
        pltpu.async_copy(rows_v.at[b], acc_sh.at[dst_v.at[j]],
                         ssem.at[b], add=True)

    def s_wait(j, b):
        pltpu.make_async_copy(rows_v.at[b], acc_sh.at[dst_v.at[j]],
                              ssem.at[b]).wait()

    g_start(0, 0)
    g_start(1, 1)

    def step(i, c):
        for b in range(NBUF):
            j = i * NBUF + b
            g_wait(j, b)
            s_start(j, b)
            s_wait(j, b)
            if b == 0:
                g_start(j + NBUF, b)
            else:
                @pl.when(i < NCH // NBUF - 1)
                def _():
                    g_start(j + NBUF, b)
        return c

    lax.fori_loop(0, NCH // NBUF, step, 0)
    g_wait(NCH - 1, 0)
    s_start(NCH - 1, 0)
    s_wait(NCH - 1, 0)

    plsc.subcore_barrier()
    pltpu.sync_copy(acc_sh.at[pl.ds(sid * ROWS_PT, ROWS_PT)],
                    acc_hbm.at[cid, pl.ds(sid * ROWS_PT, ROWS_PT)])


def _mp_call(scaled, eflat, edges):
    return pl.kernel(
        _mp_body,
        out_type=jax.ShapeDtypeStruct((NC, NP, D), jnp.float32),
        mesh=_mesh(),
        scratch_types=[
            pltpu.VMEM((EPT,), jnp.int32),
            pltpu.VMEM((NCH, CHUNK), jnp.int32),
            pltpu.VMEM((NBUF, CHUNK, D), jnp.float32),
            pltpu.SemaphoreType.DMA((NBUF,)),
            pltpu.SemaphoreType.DMA((NBUF,)),
            pltpu.VMEM_SHARED((NP, D), jnp.float32),
        ],
    )(scaled, eflat, edges)


def _final_body(acc_ref, hist_ref, out_ref):
    deg = jnp.maximum(jnp.sum(hist_ref[0], axis=0), 1.0)
    norm = lax.rsqrt(deg)
    for k in range(ROWS_TC // 128):
        sl = slice(k * 128, (k + 1) * 128)
        col = jnp.broadcast_to(norm[None, sl], (128, 128)).T
        out_ref[sl, :] = (acc_ref[0, sl, :] + acc_ref[1, sl, :]) * col


def _final_call(acc, hist):
    return pl.pallas_call(
        _final_body,
        grid=(NP // ROWS_TC,),
        in_specs=[
            pl.BlockSpec((NC, ROWS_TC, D), lambda j: (0, j, 0)),
            pl.BlockSpec((1, NW, ROWS_TC), lambda j: (1, 0, j)),
        ],
        out_specs=pl.BlockSpec((ROWS_TC, D), lambda j: (j, 0)),
        out_shape=jax.ShapeDtypeStruct((NP, D), jnp.float32),
    )(acc, hist)


def kernel(u_f, v_f, edge_index):
    node_f = jnp.concatenate([u_f, v_f], axis=0)
    node_f = jnp.pad(node_f, ((0, NP - N_NODES), (0, 0)))
    eflat = edge_index.reshape(2, NW, 1, EPT)
    edges = edge_index.reshape(2, NW, NCH, CHUNK)
    hist = _hist_call(eflat).reshape(2, NW, NP)
    scaled = _scale_call(hist, node_f)
    acc = _mp_call(scaled, eflat, edges)
    out = _final_call(acc, hist)
    return out[:N_NODES]


# speedup vs baseline: 10.5909x; 1.0005x over previous
import functools

import jax
import jax.numpy as jnp
from jax import lax
from jax.experimental import pallas as pl
from jax.experimental.pallas import tpu as pltpu
from jax.experimental.pallas import tpu_sc as plsc

N_NODES = 10000
NP = 10240
E = 320000
D = 128
NC, NS, L = 2, 16, 16
NW = NC * NS
EPT = E // NW
CHUNK = 80
NCH = EPT // CHUNK
NCH_H = EPT // L
ROWS_PT = NP // NS

_mesh = functools.partial(
    plsc.VectorSubcoreMesh, core_axis_name="c", subcore_axis_name="s")


def _hist_body(edges_hbm, hist_hbm, src_v, dst_v, hout_v, hin_v):
    cid = lax.axis_index("c")
    sid = lax.axis_index("s")
    wid = sid * NC + cid
    pltpu.sync_copy(edges_hbm.at[0, wid, 0], src_v)
    pltpu.sync_copy(edges_hbm.at[1, wid, 0], dst_v)

    zeros16 = jnp.zeros((L,), jnp.float32)
    ones16 = jnp.ones((L,), jnp.float32)

    def zero(i, c):
        hout_v[pl.ds(i * L, L)] = zeros16
        hin_v[pl.ds(i * L, L)] = zeros16
        return c

    lax.fori_loop(0, NP // L, zero, 0)

    def accum(r, c):
        sidx = src_v[pl.ds(r * L, L)]
        plsc.addupdate_scatter(hout_v, [sidx], ones16)
        didx = dst_v[pl.ds(r * L, L)]
        plsc.addupdate_scatter(hin_v, [didx], ones16)
        return c

    lax.fori_loop(0, NCH_H, accum, 0)

    pltpu.sync_copy(hout_v, hist_hbm.at[0, wid, 0])
    pltpu.sync_copy(hin_v, hist_hbm.at[1, wid, 0])


def _hist_call(edges):
    return pl.kernel(
        _hist_body,
        out_type=jax.ShapeDtypeStruct((2, NW, 1, NP), jnp.float32),
        mesh=_mesh(),
        scratch_types=[
            pltpu.VMEM((EPT,), jnp.int32),
            pltpu.VMEM((EPT,), jnp.int32),
            pltpu.VMEM((NP,), jnp.float32),
            pltpu.VMEM((NP,), jnp.float32),
        ],
        compiler_params=pltpu.CompilerParams(needs_layout_passes=False),
    )(edges)


ROWS_TC = 1024


def _scale_body(hist_ref, nf_ref, out_ref):
    deg = jnp.maximum(jnp.sum(hist_ref[0], axis=0), 1.0)
    norm = lax.rsqrt(deg)
    for k in range(ROWS_TC // 128):
        sl = slice(k * 128, (k + 1) * 128)
        col = jnp.broadcast_to(norm[None, sl], (128, 128)).T
        out_ref[sl, :] = nf_ref[sl, :] * col


def _scale_call(hist, node_f):
    return pl.pallas_call(
        _scale_body,
        grid=(NP // ROWS_TC,),
        in_specs=[
            pl.BlockSpec((2, NW, ROWS_TC), lambda j: (0, 0, j)),
            pl.BlockSpec((ROWS_TC, D), lambda j: (j, 0)),
        ],
        out_specs=pl.BlockSpec((ROWS_TC, D), lambda j: (j, 0)),
        out_shape=jax.ShapeDtypeStruct((NP, D), jnp.float32),
    )(hist, node_f)


NBUF = 2


def _mp_body(scaled_hbm, eflat_hbm, edges_hbm, acc_hbm, src_v, dst_v, rows_v,
             gsem, ssem, acc_sh):
    cid = lax.axis_index("c")
    sid = lax.axis_index("s")
    wid = sid * NC + cid
    pltpu.sync_copy(eflat_hbm.at[0, wid, 0], src_v)
    pltpu.sync_copy(edges_hbm.at[1, wid], dst_v)

    zeros16 = jnp.zeros((L,), jnp.float32)

    def zero(r, c):
        for k in range(D // L):
            rows_v[0, r, pl.ds(k * L, L)] = zeros16
        return c

    lax.fori_loop(0, CHUNK, zero, 0)
    for b in range(ROWS_PT // CHUNK):
        pltpu.sync_copy(
            rows_v.at[0], acc_sh.at[pl.ds(sid * ROWS_PT + b * CHUNK, CHUNK)])
    plsc.subcore_barrier()

    def g_start(j, b):
        pltpu.async_copy(scaled_hbm.at[src_v.at[pl.ds(j * CHUNK, CHUNK)]],
                         rows_v.at[b], gsem.at[b])

    def g_wait(j, b):
        pltpu.make_async_copy(scaled_hbm.at[src_v.at[pl.ds(j * CHUNK, CHUNK)]],
                              rows_v.at[b], gsem.at[b]).wait()

    def s_start(j, b):
        pltpu.async_copy(rows_v.at[b], acc_sh.at[dst_v.at[j]],
                         ssem.at[b], add=True)

    def s_wait(j, b):
        pltpu.make_async_copy(rows_v.at[b], acc_sh.at[dst_v.at[j]],
                              ssem.at[b]).wait()

    for b in range(NBUF):
        g_start(b, b)

    def step(i, c):
        for b in range(NBUF):
            j = i * NBUF + b
            g_wait(j, b)
            s_start(j, b)
            s_wait(j, b)

            @pl.when(j + NBUF < NCH)
            def _():
                g_start(j + NBUF, b)
        return c

    lax.fori_loop(0, NCH // NBUF, step, 0)
    for j in range((NCH // NBUF) * NBUF, NCH):
        b = j % NBUF
        g_wait(j, b)
        s_start(j, b)
        s_wait(j, b)

    plsc.subcore_barrier()
    pltpu.sync_copy(acc_sh.at[pl.ds(sid * ROWS_PT, ROWS_PT)],
                    acc_hbm.at[cid, pl.ds(sid * ROWS_PT, ROWS_PT)])


def _mp_call(scaled, eflat, edges):
    return pl.kernel(
        _mp_body,
        out_type=jax.ShapeDtypeStruct((NC, NP, D), jnp.float32),
        mesh=_mesh(),
        scratch_types=[
            pltpu.VMEM((EPT,), jnp.int32),
            pltpu.VMEM((NCH, CHUNK), jnp.int32),
            pltpu.VMEM((NBUF, CHUNK, D), jnp.float32),
            pltpu.SemaphoreType.DMA((NBUF,)),
            pltpu.SemaphoreType.DMA((NBUF,)),
            pltpu.VMEM_SHARED((NP, D), jnp.float32),
        ],
    )(scaled, eflat, edges)


def _final_body(acc_ref, hist_ref, out_ref):
    deg = jnp.maximum(jnp.sum(hist_ref[0], axis=0), 1.0)
    norm = lax.rsqrt(deg)
    for k in range(ROWS_TC // 128):
        sl = slice(k * 128, (k + 1) * 128)
        col = jnp.broadcast_to(norm[None, sl], (128, 128)).T
        out_ref[sl, :] = (acc_ref[0, sl, :] + acc_ref[1, sl, :]) * col


def _final_call(acc, hist):
    return pl.pallas_call(
        _final_body,
        grid=(NP // ROWS_TC,),
        in_specs=[
            pl.BlockSpec((NC, ROWS_TC, D), lambda j: (0, j, 0)),
            pl.BlockSpec((1, NW, ROWS_TC), lambda j: (1, 0, j)),
        ],
        out_specs=pl.BlockSpec((ROWS_TC, D), lambda j: (j, 0)),
        out_shape=jax.ShapeDtypeStruct((NP, D), jnp.float32),
    )(acc, hist)


def kernel(u_f, v_f, edge_index):
    node_f = jnp.concatenate([u_f, v_f], axis=0)
    node_f = jnp.pad(node_f, ((0, NP - N_NODES), (0, 0)))
    eflat = edge_index.reshape(2, NW, 1, EPT)
    edges = edge_index.reshape(2, NW, NCH, CHUNK)
    hist = _hist_call(eflat).reshape(2, NW, NP)
    scaled = _scale_call(hist, node_f)
    acc = _mp_call(scaled, eflat, edges)
    out = _final_call(acc, hist)
    return out[:N_NODES]
